# Initial kernel scaffold; baseline (speedup 1.0000x reference)
import jax
import jax.numpy as jnp
from jax.experimental import pallas as pl


def kernel(input, edge_index, W, a):
    raise NotImplementedError("write your pallas kernel here")


# speedup vs baseline: 4.5147x; 4.5147x over previous
import functools

import jax
import jax.numpy as jnp
from jax import lax
from jax.experimental import pallas as pl
from jax.experimental.pallas import tpu as pltpu
from jax.experimental.pallas import tpu_sc as plsc

N = 10000
E = 320000
D = 128
C = 128
NCHUNK = E // C
NCORE = 2
NSUB = 16
NW = NCORE * NSUB
NITER = (NCHUNK + NW - 1) // NW
NPAD = 10240
RPT = NPAD // NSUB
NRS = NPAD
RSPT = NRS // NSUB


def _pre_body(x_ref, w_ref, wuv_ref, h_ref, uv_ref):
    h = jnp.dot(x_ref[...], w_ref[...], preferred_element_type=jnp.float32)
    h_ref[...] = h
    uv_ref[...] = jnp.dot(h, wuv_ref[...], preferred_element_type=jnp.float32)


def _combine_body(hp_ref, rs_ref, out_ref):
    hp = hp_ref[0] + hp_ref[1]
    rs = rs_ref[0] + rs_ref[1]
    out_ref[...] = hp / (rs + 1e-16)


def _sc_body(h_hbm, src_hbm, dst_hbm, u_hbm, v_hbm, a3_hbm, a4_hbm,
             zrow_hbm, zrs_hbm,
             hp_out, rs_out,
             a3_v, a4_v, src_v, dst_v, hs_v, hd_v, esc_v, accb_v, uu_v, vv_v,
             shp, srs, sem0, sem1):
    cid = lax.axis_index("c")
    sid = lax.axis_index("s")
    w = sid * NCORE + cid

    if True:
        pltpu.sync_copy(zrow_hbm, shp.at[pl.ds(sid * RPT, RPT)])
        pltpu.sync_copy(zrs_hbm, srs.at[pl.ds(sid * RSPT, RSPT)])
        pltpu.sync_copy(a3_hbm, a3_v)
        pltpu.sync_copy(a4_hbm, a4_v)
        plsc.subcore_barrier()

        lane = lax.iota(jnp.int32, 16)
        a3b = [a3_v[pl.ds(j * 16, 16)] for j in range(D // 16)]
        a4b = [a4_v[pl.ds(j * 16, 16)] for j in range(D // 16)]

        def chunk_body(i, carry):
            chunk = w + i * NW

            @pl.when(chunk < NCHUNK)
            def _():
                off = chunk * C
                pltpu.sync_copy(src_hbm.at[pl.ds(off, C)], src_v)
                pltpu.sync_copy(dst_hbm.at[pl.ds(off, C)], dst_v)
                g1 = pltpu.async_copy(h_hbm.at[src_v], hs_v, sem0)
                g2 = pltpu.async_copy(h_hbm.at[dst_v], hd_v, sem1)
                g3 = pltpu.async_copy(u_hbm.at[src_v], uu_v, sem0)
                g4 = pltpu.async_copy(v_hbm.at[dst_v], vv_v, sem1)
                g1.wait()
                g2.wait()
                g3.wait()
                g4.wait()

                def ebody(e, c2):
                    acc = jnp.zeros((16,), jnp.float32)
                    for j in range(D // 16):
                        sl = pl.ds(j * 16, 16)
                        hs_j = hs_v[e, sl]
                        hd_j = hd_v[e, sl]
                        acc = acc + a3b[j] * jnp.abs(hs_j + hd_j)
                        acc = acc + a4b[j] * jnp.abs(hs_j - hd_j)
                    accb_v[e, :] = acc
                    return c2

                lax.fori_loop(0, C, ebody, 0, unroll=2)

                for g in range(C // 16):
                    row16 = lane + (g * 16)
                    u16 = uu_v[pl.ds(g * 16, 16)]
                    v16 = vv_v[pl.ds(g * 16, 16)]
                    acc = jnp.zeros((16,), jnp.float32)
                    for j in range(16):
                        jj = jnp.full((16,), j, jnp.int32)
                        acc = acc + plsc.load_gather(accb_v, [row16, jj])
                    score = u16 + v16 + 0.4 * acc
                    e16 = jnp.exp(-(0.6 * score + 0.4 * jnp.abs(score)))
                    esc_v[pl.ds(g * 16, 16)] = e16

                def scale_body(e, c2):
                    ee = jnp.full((16,), 0, jnp.int32) + e
                    sv = plsc.load_gather(esc_v, [ee])
                    for j in range(D // 16):
                        sl = pl.ds(j * 16, 16)
                        hd_v[e, sl] = hd_v[e, sl] * sv
                    return c2

                lax.fori_loop(0, C, scale_body, 0, unroll=2)

                pltpu.sync_copy(hd_v, shp.at[src_v], add=True)
                pltpu.sync_copy(esc_v, srs.at[src_v], add=True)

            return carry

        lax.fori_loop(0, NITER, chunk_body, 0)

        plsc.subcore_barrier()
        pltpu.sync_copy(shp.at[pl.ds(sid * RPT, RPT)],
                        hp_out.at[cid, pl.ds(sid * RPT, RPT)])
        pltpu.sync_copy(srs.at[pl.ds(sid * RSPT, RSPT)],
                        rs_out.at[cid, pl.ds(sid * RSPT, RSPT)])


@jax.jit
def kernel(input, edge_index, W, a):
    x = input
    src = edge_index[0]
    dst = edge_index[1]
    a1 = a[0, :D]
    a2 = a[0, D:2 * D]
    a3 = a[0, 2 * D:3 * D]
    a4 = a[0, 3 * D:]
    wu = a1 + 0.6 * (a3 + a4)
    wv = a2 + 0.6 * (a3 - a4)
    wuv = jnp.zeros((D, 8), jnp.float32)
    wuv = wuv.at[:, 0].set(wu).at[:, 1].set(wv)

    h, uv = pl.pallas_call(
        _pre_body,
        out_shape=[
            jax.ShapeDtypeStruct((N, D), jnp.float32),
            jax.ShapeDtypeStruct((N, 8), jnp.float32),
        ],
    )(x, W, wuv)
    u = uv[:, 0]
    v = uv[:, 1]

    zrow = jnp.zeros((RPT, D), jnp.float32)
    zrs = jnp.zeros((RSPT,), jnp.float32)

    sc = pl.kernel(
        _sc_body,
        out_type=(
            jax.ShapeDtypeStruct((NCORE, NPAD, D), jnp.float32),
            jax.ShapeDtypeStruct((NCORE, NRS), jnp.float32),
        ),
        mesh=plsc.VectorSubcoreMesh(core_axis_name="c", subcore_axis_name="s"),
        compiler_params=pltpu.CompilerParams(
            needs_layout_passes=False, use_tc_tiling_on_sc=False),
        scratch_types=[
            pltpu.VMEM((D,), jnp.float32),
            pltpu.VMEM((D,), jnp.float32),
            pltpu.VMEM((C,), jnp.int32),
            pltpu.VMEM((C,), jnp.int32),
            pltpu.VMEM((C, D), jnp.float32),
            pltpu.VMEM((C, D), jnp.float32),
            pltpu.VMEM((C,), jnp.float32),
            pltpu.VMEM((C, 16), jnp.float32),
            pltpu.VMEM((C,), jnp.float32),
            pltpu.VMEM((C,), jnp.float32),
            pltpu.VMEM_SHARED((NPAD, D), jnp.float32),
            pltpu.VMEM_SHARED((NRS,), jnp.float32),
            pltpu.SemaphoreType.DMA,
            pltpu.SemaphoreType.DMA,
        ],
    )
    hp_part, rs_part = sc(h, src, dst, u, v, a3, a4, zrow, zrs)

    hp_part = hp_part[:, :N]
    rs2 = rs_part[:, :N, None]
    out = pl.pallas_call(
        _combine_body,
        out_shape=jax.ShapeDtypeStruct((N, D), jnp.float32),
    )(hp_part, rs2)
    return out


# speedup vs baseline: 5.8763x; 1.3016x over previous
import jax
import jax.numpy as jnp
from jax import lax
from jax.experimental import pallas as pl
from jax.experimental.pallas import tpu as pltpu
from jax.experimental.pallas import tpu_sc as plsc

N = 10000
E = 320000
D = 128
C = 80
G = C // 16
NCORE = 2
NSUB = 16
NW = NCORE * NSUB
EPW = E // NW
CPW = EPW // C
NPAD = 10240
RPT = NPAD // NSUB


def _pre_body(x_ref, w_ref, wuv_ref, h_ref, uv_ref):
    h = jnp.dot(x_ref[...], w_ref[...], preferred_element_type=jnp.float32)
    h_ref[...] = h
    uv_ref[...] = jnp.dot(h, wuv_ref[...], preferred_element_type=jnp.float32,
                          precision=jax.lax.Precision.HIGHEST)


def _combine_body(hp_ref, rs_ref, out_ref):
    hp = hp_ref[0] + hp_ref[1]
    rs = rs_ref[0] + rs_ref[1]
    out_ref[...] = hp / (rs + 1e-16)


def _sc_body(h_hbm, src_hbm, dst_hbm, u_hbm, v_hbm, a3_hbm, a4_hbm,
             zrow_hbm, zrs_hbm,
             hp_out, rs_out,
             a3_v, a4_v,
             src_v0, src_v1, dst_v0, dst_v1, srcs_v0, srcs_v1,
             uu_v0, uu_v1, vv_v0, vv_v1,
             hs_v0, hs_v1, hd_v0, hd_v1, esc_v0, esc_v1, accb_v,
             shp, srs,
             semg0, semg1, sems0, sems1, semi0, semi1):
    src_v = [src_v0, src_v1]
    dst_v = [dst_v0, dst_v1]
    srcs_v = [srcs_v0, srcs_v1]
    uu_v = [uu_v0, uu_v1]
    vv_v = [vv_v0, vv_v1]
    hs_v = [hs_v0, hs_v1]
    hd_v = [hd_v0, hd_v1]
    esc_v = [esc_v0, esc_v1]
    semg = [semg0, semg1]
    sems = [sems0, sems1]
    semi = [semi0, semi1]

    cid = lax.axis_index("c")
    sid = lax.axis_index("s")
    w = sid * NCORE + cid
    base = w * EPW

    pltpu.sync_copy(zrow_hbm, shp.at[pl.ds(sid * RPT, RPT)])
    pltpu.sync_copy(zrs_hbm, srs.at[pl.ds(sid * RPT, RPT)])
    pltpu.sync_copy(a3_hbm, a3_v)
    pltpu.sync_copy(a4_hbm, a4_v)
    plsc.subcore_barrier()

    lane = lax.iota(jnp.int32, 16)
    a3b = [a3_v[pl.ds(j * 16, 16)] for j in range(D // 16)]
    a4b = [a4_v[pl.ds(j * 16, 16)] for j in range(D // 16)]

    def issue_idx(c, s):
        off = base + c * C
        pltpu.async_copy(src_hbm.at[pl.ds(off, C)], src_v[s], semi[s])
        pltpu.async_copy(dst_hbm.at[pl.ds(off, C)], dst_v[s], semi[s])

    def wait_idx(c, s):
        off = base + c * C
        pltpu.make_async_copy(src_hbm.at[pl.ds(off, C)], src_v[s],
                              semi[s]).wait()
        pltpu.make_async_copy(dst_hbm.at[pl.ds(off, C)], dst_v[s],
                              semi[s]).wait()

    def issue_gathers(s):
        pltpu.async_copy(h_hbm.at[src_v[s]], hs_v[s], semg[s])
        pltpu.async_copy(h_hbm.at[dst_v[s]], hd_v[s], semg[s])
        pltpu.async_copy(u_hbm.at[src_v[s]], uu_v[s], semg[s])
        pltpu.async_copy(v_hbm.at[dst_v[s]], vv_v[s], semg[s])

    def wait_gathers(s):
        pltpu.make_async_copy(h_hbm.at[src_v[s]], hs_v[s], semg[s]).wait()
        pltpu.make_async_copy(h_hbm.at[dst_v[s]], hd_v[s], semg[s]).wait()
        pltpu.make_async_copy(u_hbm.at[src_v[s]], uu_v[s], semg[s]).wait()
        pltpu.make_async_copy(v_hbm.at[dst_v[s]], vv_v[s], semg[s]).wait()

    def issue_scatter(s):
        pltpu.async_copy(hd_v[s], shp.at[srcs_v[s]], sems[s], add=True)
        pltpu.async_copy(esc_v[s], srs.at[srcs_v[s]], sems[s], add=True)

    def wait_scatter(s):
        pltpu.make_async_copy(hd_v[s], shp.at[srcs_v[s]], sems[s]).wait()
        pltpu.make_async_copy(esc_v[s], srs.at[srcs_v[s]], sems[s]).wait()

    def process(c, s, drain_prev, last):
        o = 1 - s
        wait_gathers(s)
        for g in range(G):
            sl = pl.ds(g * 16, 16)
            srcs_v[s][sl] = src_v[s][sl]
        if not last:
            @pl.when(c + 2 < CPW)
            def _():
                issue_idx(c + 2, s)

        def ebody(e, c2):
            acc = jnp.zeros((16,), jnp.float32)
            for j in range(D // 16):
                sl = pl.ds(j * 16, 16)
                hs_j = hs_v[s][e, sl]
                hd_j = hd_v[s][e, sl]
                acc = acc + a3b[j] * jnp.abs(hs_j + hd_j)
                acc = acc + a4b[j] * jnp.abs(hs_j - hd_j)
            accb_v[e, :] = acc
            return c2

        lax.fori_loop(0, C, ebody, 0, unroll=2)

        for g in range(G):
            row16 = lane + (g * 16)
            u16 = uu_v[s][pl.ds(g * 16, 16)]
            v16 = vv_v[s][pl.ds(g * 16, 16)]
            acc = jnp.zeros((16,), jnp.float32)
            for j in range(16):
                jj = jnp.full((16,), j, jnp.int32)
                acc = acc + plsc.load_gather(accb_v, [row16, jj])
            score = u16 + v16 + 0.4 * acc
            e16 = jnp.exp(-(0.6 * score + 0.4 * jnp.abs(score)))
            esc_v[s][pl.ds(g * 16, 16)] = e16

        if not last:
            if drain_prev:
                wait_scatter(o)
            wait_idx(c + 1, o)
            issue_gathers(o)

        def scale_body(e, c2):
            ee = jnp.full((16,), 0, jnp.int32) + e
            sv = plsc.load_gather(esc_v[s], [ee])
            for j in range(D // 16):
                sl = pl.ds(j * 16, 16)
                hd_v[s][e, sl] = hd_v[s][e, sl] * sv
            return c2

        lax.fori_loop(0, C, scale_body, 0, unroll=2)

        issue_scatter(s)

    issue_idx(0, 0)
    wait_idx(0, 0)
    issue_gathers(0)
    issue_idx(1, 1)

    def pair_body(j, carry):
        process(2 * j, 0, drain_prev=True, last=False)
        process(2 * j + 1, 1, drain_prev=True, last=False)
        return carry

    process(0, 0, drain_prev=False, last=False)
    process(1, 1, drain_prev=True, last=False)
    lax.fori_loop(1, (CPW - 1) // 2, pair_body, 0)
    process(CPW - 1, 0, drain_prev=True, last=True)
    wait_scatter(1)
    wait_scatter(0)

    plsc.subcore_barrier()
    pltpu.sync_copy(shp.at[pl.ds(sid * RPT, RPT)],
                    hp_out.at[cid, pl.ds(sid * RPT, RPT)])
    pltpu.sync_copy(srs.at[pl.ds(sid * RPT, RPT)],
                    rs_out.at[cid, pl.ds(sid * RPT, RPT)])


@jax.jit
def kernel(input, edge_index, W, a):
    x = input
    src = edge_index[0]
    dst = edge_index[1]
    a1 = a[0, :D]
    a2 = a[0, D:2 * D]
    a3 = a[0, 2 * D:3 * D]
    a4 = a[0, 3 * D:]
    wu = a1 + 0.6 * (a3 + a4)
    wv = a2 + 0.6 * (a3 - a4)
    wuv = jnp.zeros((D, 8), jnp.float32)
    wuv = wuv.at[:, 0].set(wu).at[:, 1].set(wv)

    h, uv = pl.pallas_call(
        _pre_body,
        out_shape=[
            jax.ShapeDtypeStruct((N, D), jnp.float32),
            jax.ShapeDtypeStruct((N, 8), jnp.float32),
        ],
    )(x, W, wuv)
    u = uv[:, 0]
    v = uv[:, 1]

    zrow = jnp.zeros((RPT, D), jnp.float32)
    zrs = jnp.zeros((RPT,), jnp.float32)

    sc = pl.kernel(
        _sc_body,
        out_type=(
            jax.ShapeDtypeStruct((NCORE, NPAD, D), jnp.float32),
            jax.ShapeDtypeStruct((NCORE, NPAD), jnp.float32),
        ),
        mesh=plsc.VectorSubcoreMesh(core_axis_name="c", subcore_axis_name="s"),
        compiler_params=pltpu.CompilerParams(
            needs_layout_passes=False, use_tc_tiling_on_sc=False),
        scratch_types=[
            pltpu.VMEM((D,), jnp.float32),
            pltpu.VMEM((D,), jnp.float32),
            pltpu.VMEM((C,), jnp.int32),
            pltpu.VMEM((C,), jnp.int32),
            pltpu.VMEM((C,), jnp.int32),
            pltpu.VMEM((C,), jnp.int32),
            pltpu.VMEM((C,), jnp.int32),
            pltpu.VMEM((C,), jnp.int32),
            pltpu.VMEM((C,), jnp.float32),
            pltpu.VMEM((C,), jnp.float32),
            pltpu.VMEM((C,), jnp.float32),
            pltpu.VMEM((C,), jnp.float32),
            pltpu.VMEM((C, D), jnp.float32),
            pltpu.VMEM((C, D), jnp.float32),
            pltpu.VMEM((C, D), jnp.float32),
            pltpu.VMEM((C, D), jnp.float32),
            pltpu.VMEM((C,), jnp.float32),
            pltpu.VMEM((C,), jnp.float32),
            pltpu.VMEM((C, 16), jnp.float32),
            pltpu.VMEM_SHARED((NPAD, D), jnp.float32),
            pltpu.VMEM_SHARED((NPAD,), jnp.float32),
            pltpu.SemaphoreType.DMA,
            pltpu.SemaphoreType.DMA,
            pltpu.SemaphoreType.DMA,
            pltpu.SemaphoreType.DMA,
            pltpu.SemaphoreType.DMA,
            pltpu.SemaphoreType.DMA,
        ],
    )
    hp_part, rs_part = sc(h, src, dst, u, v, a3, a4, zrow, zrs)

    hp_part = hp_part[:, :N]
    rs2 = rs_part[:, :N, None]
    out = pl.pallas_call(
        _combine_body,
        out_shape=jax.ShapeDtypeStruct((N, D), jnp.float32),
    )(hp_part, rs2)
    return out
